# Initial kernel scaffold; baseline (speedup 1.0000x reference)
import jax
import jax.numpy as jnp
from jax.experimental import pallas as pl


def kernel(x, edge_index, W1, b1, W2, b2, W3, b3):
    raise NotImplementedError("write your pallas kernel here")


# speedup vs baseline: 4.5506x; 4.5506x over previous
import functools

import jax
import jax.numpy as jnp
from jax import lax
from jax.experimental import pallas as pl
from jax.experimental.pallas import tpu as pltpu
from jax.experimental.pallas import tpu_sc as plsc

N = 10000
E = 320000
F = 128
H = 128
C = 16

NW = 32
K = 128
J = 79
EPAD = NW * J * K - E
NP = 10240
RPT = NP // 16
BLK = 1000


def _mm_kernel(x_ref, w_ref, o_ref):
    o_ref[...] = jnp.dot(x_ref[...], w_ref[...],
                         preferred_element_type=jnp.float32)


def _comb_mm_kernel(p_ref, b_ref, w_ref, o_ref):
    agg = p_ref[0] + p_ref[1]
    h = jnp.maximum(agg + b_ref[...], 0.0)
    o_ref[...] = jnp.dot(h, w_ref[...], preferred_element_type=jnp.float32)


def _final_kernel(p_ref, b_ref, w_ref, b3_ref, o_ref):
    agg = p_ref[0] + p_ref[1]
    h = jnp.maximum(agg + b_ref[...], 0.0)
    logits = jnp.dot(h, w_ref[...], preferred_element_type=jnp.float32)
    logits = logits + b3_ref[...]
    m = jnp.max(logits, axis=1, keepdims=True)
    shifted = logits - m
    o_ref[...] = shifted - jnp.log(
        jnp.sum(jnp.exp(shifted), axis=1, keepdims=True))


def _sc_segsum(support, srcp, dstp, zeros):
    mesh = plsc.VectorSubcoreMesh(core_axis_name="c", subcore_axis_name="s")

    @functools.partial(
        pl.kernel,
        out_type=jax.ShapeDtypeStruct((2, NP, H), jnp.float32),
        mesh=mesh,
        scratch_types=[
            pltpu.VMEM((J, K), jnp.int32),
            pltpu.VMEM((J, K), jnp.int32),
            pltpu.VMEM((K, H), jnp.float32),
            pltpu.VMEM_SHARED((NP, H), jnp.float32),
            pltpu.SemaphoreType.DMA,
        ],
    )
    def seg_kernel(zeros_hbm, support_hbm, src_hbm, dst_hbm, out_hbm,
                   srcv, dstv, rows, acc, sem):
        c = lax.axis_index("c")
        s = lax.axis_index("s")
        wid = s * 2 + c
        pltpu.sync_copy(zeros_hbm.at[pl.ds(s * RPT, RPT)],
                        acc.at[pl.ds(s * RPT, RPT)])
        pltpu.sync_copy(src_hbm.at[wid], srcv)
        pltpu.sync_copy(dst_hbm.at[wid], dstv)
        plsc.subcore_barrier()

        def body(j, carry):
            pltpu.async_copy(support_hbm.at[srcv.at[j]], rows, sem).wait()
            pltpu.sync_copy(rows, acc.at[dstv.at[j]], add=True)
            return carry

        lax.fori_loop(0, J, body, 0)
        plsc.subcore_barrier()
        pltpu.sync_copy(acc.at[pl.ds(s * RPT, RPT)],
                        out_hbm.at[c, pl.ds(s * RPT, RPT)])

    return seg_kernel(zeros, support, srcp, dstp)


def _tc_matmul(x, w):
    return pl.pallas_call(
        _mm_kernel,
        grid=(N // BLK,),
        in_specs=[
            pl.BlockSpec((BLK, F), lambda i: (i, 0)),
            pl.BlockSpec((F, H), lambda i: (0, 0)),
        ],
        out_specs=pl.BlockSpec((BLK, H), lambda i: (i, 0)),
        out_shape=jax.ShapeDtypeStruct((N, H), jnp.float32),
    )(x, w)


def _tc_comb_matmul(part, b, w):
    return pl.pallas_call(
        _comb_mm_kernel,
        grid=(N // BLK,),
        in_specs=[
            pl.BlockSpec((2, BLK, H), lambda i: (0, i, 0)),
            pl.BlockSpec((1, H), lambda i: (0, 0)),
            pl.BlockSpec((H, H), lambda i: (0, 0)),
        ],
        out_specs=pl.BlockSpec((BLK, H), lambda i: (i, 0)),
        out_shape=jax.ShapeDtypeStruct((N, H), jnp.float32),
    )(part, b, w)


def _tc_final(part, b, w, b3):
    return pl.pallas_call(
        _final_kernel,
        grid=(N // BLK,),
        in_specs=[
            pl.BlockSpec((2, BLK, H), lambda i: (0, i, 0)),
            pl.BlockSpec((1, H), lambda i: (0, 0)),
            pl.BlockSpec((H, C), lambda i: (0, 0)),
            pl.BlockSpec((1, C), lambda i: (0, 0)),
        ],
        out_specs=pl.BlockSpec((BLK, C), lambda i: (i, 0)),
        out_shape=jax.ShapeDtypeStruct((N, C), jnp.float32),
    )(part, b, w, b3)


def kernel(x, edge_index, W1, b1, W2, b2, W3, b3):
    f32 = jnp.float32
    W1p = jnp.zeros((F, H), f32).at[:, :W1.shape[1]].set(W1)
    W2p = jnp.zeros((H, H), f32).at[:W2.shape[0], :W2.shape[1]].set(W2)
    W3p = jnp.zeros((H, C), f32).at[:W3.shape[0], :].set(W3)
    b1p = jnp.zeros((1, H), f32).at[0, :b1.shape[0]].set(b1)
    b2p = jnp.zeros((1, H), f32).at[0, :b2.shape[0]].set(b2)
    b3p = b3.reshape(1, C)

    src = edge_index[0]
    dst = edge_index[1]
    srcp = jnp.concatenate(
        [src, jnp.zeros((EPAD,), jnp.int32)]).reshape(NW, J, K)
    dstp = jnp.concatenate(
        [dst, jnp.full((EPAD,), N, jnp.int32)]).reshape(NW, J, K)
    zeros = jnp.zeros((NP, H), f32)

    support1 = _tc_matmul(x, W1p)
    part1 = _sc_segsum(support1, srcp, dstp, zeros)
    support2 = _tc_comb_matmul(part1, b1p, W2p)
    part2 = _sc_segsum(support2, srcp, dstp, zeros)
    return _tc_final(part2, b2p, W3p, b3p)
